# baseline (speedup 1.0000x reference)
import jax
import jax.numpy as jnp
from jax.experimental import pallas as pl
from jax.experimental.pallas import tpu as pltpu


def _vq_kernel(x_ref, cb_ref, out_ref, cbsq_ref):
    i = pl.program_id(0)
    cb = cb_ref[...]

    @pl.when(i == 0)
    def _():
        cbsq_ref[...] = jnp.sum(cb * cb, axis=1)[None, :]

    x = x_ref[...]
    in_sq = jnp.sum(x * x, axis=1, keepdims=True)
    mm = jax.lax.dot_general(
        x * (-2.0), cb,
        dimension_numbers=(((1,), (1,)), ((), ())),
        preferred_element_type=jnp.float32,
    )
    dist = (cbsq_ref[...] + in_sq) + mm
    out_ref[...] = jnp.argmin(dist, axis=1).astype(jnp.int32)[None, None, :]


def kernel(z_e_x, codebook):
    Bv, Nv, D = z_e_x.shape
    K = codebook.shape[0]
    M = Bv * Nv
    flat = z_e_x.reshape(M, D).astype(jnp.float32)
    cb = codebook.astype(jnp.float32)

    BM = 576
    grid = (M // BM,)
    idx = pl.pallas_call(
        _vq_kernel,
        grid=grid,
        in_specs=[
            pl.BlockSpec((BM, D), lambda i: (i, 0)),
            pl.BlockSpec((K, D), lambda i: (0, 0)),
        ],
        out_specs=pl.BlockSpec((1, 1, BM), lambda i: (i, 0, 0)),
        out_shape=jax.ShapeDtypeStruct((M // BM, 1, BM), jnp.int32),
        scratch_shapes=[pltpu.VMEM((1, K), jnp.float32)],
    )(flat, cb)
    return idx.reshape(Bv, Nv)


# speedup vs baseline: 1.0268x; 1.0268x over previous
import jax
import jax.numpy as jnp
from jax.experimental import pallas as pl
from jax.experimental.pallas import tpu as pltpu


def _vq_kernel(x_ref, cb_ref, out_ref, cbsq_ref):
    i = pl.program_id(0)
    cb = cb_ref[...]

    @pl.when(i == 0)
    def _():
        cbsq_ref[...] = jnp.sum(cb * cb, axis=1, keepdims=True)

    x = x_ref[...]
    in_sq = jnp.sum(x * x, axis=1)[None, :]
    mmT = jax.lax.dot_general(
        cb, x * (-2.0),
        dimension_numbers=(((1,), (1,)), ((), ())),
        preferred_element_type=jnp.float32,
    )
    dist = (cbsq_ref[...] + in_sq) + mmT
    out_ref[...] = jnp.argmin(dist, axis=0).astype(jnp.int32)[None, None, :]


def kernel(z_e_x, codebook):
    Bv, Nv, D = z_e_x.shape
    K = codebook.shape[0]
    M = Bv * Nv
    flat = z_e_x.reshape(M, D).astype(jnp.float32)
    cb = codebook.astype(jnp.float32)

    BM = 576
    grid = (M // BM,)
    idx = pl.pallas_call(
        _vq_kernel,
        grid=grid,
        in_specs=[
            pl.BlockSpec((BM, D), lambda i: (i, 0)),
            pl.BlockSpec((K, D), lambda i: (0, 0)),
        ],
        out_specs=pl.BlockSpec((1, 1, BM), lambda i: (i, 0, 0)),
        out_shape=jax.ShapeDtypeStruct((M // BM, 1, BM), jnp.int32),
        scratch_shapes=[pltpu.VMEM((K, 1), jnp.float32)],
    )(flat, cb)
    return idx.reshape(Bv, Nv)


# speedup vs baseline: 1.1551x; 1.1249x over previous
import jax
import jax.numpy as jnp
from jax.experimental import pallas as pl
from jax.experimental.pallas import tpu as pltpu

_KC = 1024


def _vq_kernel(x_ref, cb_ref, out_ref):
    x = x_ref[...]
    xm2 = x * (-2.0)
    in_sq = jnp.sum(x * x, axis=1)[None, :]
    K = cb_ref.shape[0]
    M = x.shape[0]

    run_min = None
    run_idx = None
    for c in range(K // _KC):
        cbc = cb_ref[c * _KC:(c + 1) * _KC, :]
        cbsq = jnp.sum(cbc * cbc, axis=1, keepdims=True)
        mmT = jax.lax.dot_general(
            cbc, xm2,
            dimension_numbers=(((1,), (1,)), ((), ())),
            preferred_element_type=jnp.float32,
        )
        dist = (cbsq + in_sq) + mmT
        loc_min = jnp.min(dist, axis=0)[None, :]
        loc_idx = jnp.argmin(dist, axis=0).astype(jnp.int32)[None, :] + (c * _KC)
        if run_min is None:
            run_min, run_idx = loc_min, loc_idx
        else:
            upd = loc_min < run_min
            run_min = jnp.where(upd, loc_min, run_min)
            run_idx = jnp.where(upd, loc_idx, run_idx)
    out_ref[...] = run_idx[None, :, :]


def kernel(z_e_x, codebook):
    Bv, Nv, D = z_e_x.shape
    K = codebook.shape[0]
    M = Bv * Nv
    flat = z_e_x.reshape(M, D).astype(jnp.float32)
    cb = codebook.astype(jnp.float32)

    idx = pl.pallas_call(
        _vq_kernel,
        grid=(1,),
        in_specs=[
            pl.BlockSpec((M, D), lambda i: (0, 0)),
            pl.BlockSpec((K, D), lambda i: (0, 0)),
        ],
        out_specs=pl.BlockSpec((1, 1, M), lambda i: (0, 0, 0)),
        out_shape=jax.ShapeDtypeStruct((1, 1, M), jnp.int32),
    )(flat, cb)
    return idx.reshape(Bv, Nv)
